# baseline (speedup 1.0000x reference)
import jax
import jax.numpy as jnp
from jax.experimental import pallas as pl
from jax.experimental.pallas import tpu as pltpu

N = 2048
IN_FEATURES = 128
OUT_FEATURES = 128
NUM_BANKS = 64
TOP_K = 2
TILE = 2048
NTILES = N // TILE


def _mm_kernel(sel_ref, prob_ref, x_ref, w_ref, b_ref, out_ref, wb_ref):
    t = pl.program_id(0)

    @pl.when(t == 0)
    def _():
        for b in range(NUM_BANKS):
            wb_ref[b] = w_ref[b].astype(jnp.bfloat16)

    sel = sel_ref[...]
    prob = prob_ref[...]
    banks = jax.lax.broadcasted_iota(jnp.int32, (TILE, NUM_BANKS), 1)
    p = jnp.zeros((TILE, NUM_BANKS), jnp.float32)
    for k in range(TOP_K):
        p += jnp.where(sel[:, k:k + 1] == banks, prob[:, k:k + 1], 0.0)

    x = x_ref[...].astype(jnp.bfloat16)
    pb = p.astype(jnp.bfloat16)
    bb = b_ref[...].astype(jnp.bfloat16)
    acc = jnp.dot(pb, bb,
                  preferred_element_type=jnp.float32).astype(jnp.bfloat16)
    for b in range(NUM_BANKS):
        z = jnp.dot(x, wb_ref[b], preferred_element_type=jnp.float32)
        acc = acc + pb[:, b:b + 1] * z.astype(jnp.bfloat16)
    out_ref[...] = acc.astype(jnp.float32)


def kernel(tensor, bank_selections, bank_probabilities, weights, bias):
    sel = bank_selections.astype(jnp.int32)

    out = pl.pallas_call(
        _mm_kernel,
        grid=(NTILES,),
        in_specs=[
            pl.BlockSpec((TILE, TOP_K), lambda t: (t, 0)),
            pl.BlockSpec((TILE, TOP_K), lambda t: (t, 0)),
            pl.BlockSpec((TILE, IN_FEATURES), lambda t: (t, 0)),
            pl.BlockSpec((NUM_BANKS, IN_FEATURES, OUT_FEATURES),
                         lambda t: (0, 0, 0)),
            pl.BlockSpec((NUM_BANKS, OUT_FEATURES), lambda t: (0, 0)),
        ],
        out_specs=pl.BlockSpec((TILE, OUT_FEATURES), lambda t: (t, 0)),
        out_shape=jax.ShapeDtypeStruct((N, OUT_FEATURES), jnp.float32),
        scratch_shapes=[
            pltpu.VMEM((NUM_BANKS, IN_FEATURES, OUT_FEATURES), jnp.bfloat16),
        ],
        compiler_params=pltpu.CompilerParams(
            dimension_semantics=("arbitrary",),
        ),
    )(sel, bank_probabilities, tensor, weights, bias)
    return out


# speedup vs baseline: 1.0324x; 1.0324x over previous
import jax
import jax.numpy as jnp
from jax.experimental import pallas as pl
from jax.experimental.pallas import tpu as pltpu

N = 2048
IN_FEATURES = 128
OUT_FEATURES = 128
NUM_BANKS = 64
TOP_K = 2
TILE = 1024
NTILES = N // TILE


def _mm_kernel(sel_ref, prob_ref, x_ref, w_ref, b_ref, out_ref, wb_ref):
    t = pl.program_id(0)

    @pl.when(t == 0)
    def _():
        for b in range(NUM_BANKS):
            wb_ref[b] = w_ref[b].astype(jnp.bfloat16)

    sel = sel_ref[...]
    prob = prob_ref[...]
    banks = jax.lax.broadcasted_iota(jnp.int32, (TILE, NUM_BANKS), 1)
    p = jnp.zeros((TILE, NUM_BANKS), jnp.float32)
    for k in range(TOP_K):
        p += jnp.where(sel[:, k:k + 1] == banks, prob[:, k:k + 1], 0.0)

    x = x_ref[...].astype(jnp.bfloat16)
    pb = p.astype(jnp.bfloat16)
    bb = b_ref[...].astype(jnp.bfloat16)
    acc = jnp.dot(pb, bb,
                  preferred_element_type=jnp.float32).astype(jnp.bfloat16)
    for b in range(NUM_BANKS):
        z = jnp.dot(x, wb_ref[b], preferred_element_type=jnp.float32)
        acc = acc + pb[:, b:b + 1] * z.astype(jnp.bfloat16)
    out_ref[...] = acc.astype(jnp.float32)


def kernel(tensor, bank_selections, bank_probabilities, weights, bias):
    sel = bank_selections.astype(jnp.int32)

    out = pl.pallas_call(
        _mm_kernel,
        grid=(NTILES,),
        in_specs=[
            pl.BlockSpec((TILE, TOP_K), lambda t: (t, 0)),
            pl.BlockSpec((TILE, TOP_K), lambda t: (t, 0)),
            pl.BlockSpec((TILE, IN_FEATURES), lambda t: (t, 0)),
            pl.BlockSpec((NUM_BANKS, IN_FEATURES, OUT_FEATURES),
                         lambda t: (0, 0, 0)),
            pl.BlockSpec((NUM_BANKS, OUT_FEATURES), lambda t: (0, 0)),
        ],
        out_specs=pl.BlockSpec((TILE, OUT_FEATURES), lambda t: (t, 0)),
        out_shape=jax.ShapeDtypeStruct((N, OUT_FEATURES), jnp.float32),
        scratch_shapes=[
            pltpu.VMEM((NUM_BANKS, IN_FEATURES, OUT_FEATURES), jnp.bfloat16),
        ],
        compiler_params=pltpu.CompilerParams(
            dimension_semantics=("arbitrary",),
        ),
    )(sel, bank_probabilities, tensor, weights, bias)
    return out
